# Initial kernel scaffold; baseline (speedup 1.0000x reference)
import jax
import jax.numpy as jnp
from jax.experimental import pallas as pl


def kernel(x, edge_index, edge_attr, u, batch, We1, be1, We2, be2, Mk, Mv, Wn1, bn1, Wn2, bn2, Wg1, bg1, Wg2, bg2):
    raise NotImplementedError("write your pallas kernel here")


# speedup vs baseline: 7.5750x; 7.5750x over previous
import functools

import jax
import jax.numpy as jnp
from jax import lax
from jax.experimental import pallas as pl
from jax.experimental.pallas import tpu as pltpu
from jax.experimental.pallas import tpu_sc as plsc

N = 10000
E = 320000
G = 64
VIN = 128
EIN = 16
UIN = 16
H = 128

NW = 32
EPW = E // NW
CH = 80
NCH = EPW // CH
EPC = E // 2
ZR = 624
ZTAIL = N - 16 * ZR


def _tables_body(x_ref, u_ref, batch_ref, We1_ref, be1_ref, P2_ref, Q_ref):
    x = x_ref[...]
    A = We1_ref[0:VIN, :]
    B = We1_ref[VIN:2 * VIN, :]
    D = We1_ref[2 * VIN + EIN:, :]
    uD = jnp.dot(u_ref[...], D, preferred_element_type=jnp.float32)
    gid = lax.broadcasted_iota(jnp.int32, (N, G), 1)
    onehot = (batch_ref[...] == gid).astype(jnp.float32)
    P2_ref[...] = (jnp.dot(x, A, preferred_element_type=jnp.float32)
                   + jnp.dot(onehot, uD, preferred_element_type=jnp.float32)
                   + be1_ref[...])
    Q_ref[...] = jnp.dot(x, B, preferred_element_type=jnp.float32)


def _node_tables(x, u, batch_col, We1, be1_row):
    return pl.pallas_call(
        _tables_body,
        out_shape=(jax.ShapeDtypeStruct((N, H), jnp.float32),
                   jax.ShapeDtypeStruct((N, H), jnp.float32)),
    )(x, u, batch_col, We1, be1_row)


def _gather_body(P2_hbm, Q_hbm, src_hbm, dst_hbm, out_hbm,
                 sidx, didx, bufA, bufB, semA, semB):
    wid = lax.axis_index("s") * 2 + lax.axis_index("c")
    base = wid * EPW

    def chunk(i, carry):
        off = base + i * CH
        pltpu.sync_copy(src_hbm.at[pl.ds(off, CH)], sidx)
        pltpu.sync_copy(dst_hbm.at[pl.ds(off, CH)], didx)
        cpA = pltpu.async_copy(P2_hbm.at[sidx], bufA, semA)
        cpB = pltpu.async_copy(Q_hbm.at[didx], bufB, semB)
        cpA.wait()
        cpB.wait()

        def add_row(r, c2):
            for k in range(H // 16):
                v = bufB[r, pl.ds(k * 16, 16)]
                plsc.addupdate(bufA.at[r, pl.ds(k * 16, 16)], v)
            return c2

        lax.fori_loop(0, CH, add_row, 0)
        pltpu.sync_copy(bufA, out_hbm.at[pl.ds(off, CH)])
        return carry

    lax.fori_loop(0, NCH, chunk, 0)


def _edge_gather(P2, Q, src, dst):
    mesh = plsc.VectorSubcoreMesh(core_axis_name="c", subcore_axis_name="s")
    return pl.kernel(
        _gather_body,
        out_type=jax.ShapeDtypeStruct((E, H), jnp.float32),
        mesh=mesh,
        scratch_types=[
            pltpu.VMEM((CH,), jnp.int32),
            pltpu.VMEM((CH,), jnp.int32),
            pltpu.VMEM((CH, H), jnp.float32),
            pltpu.VMEM((CH, H), jnp.float32),
            pltpu.SemaphoreType.DMA,
            pltpu.SemaphoreType.DMA,
        ],
    )(P2, Q, src, dst)


EBLK = 2000


def _edge_mlp_body(pre_ref, ea_ref, C_ref, We2_ref, be2_ref, eh_ref):
    h = jnp.maximum(
        pre_ref[...] + jnp.dot(ea_ref[...], C_ref[...],
                               preferred_element_type=jnp.float32), 0.0)
    eh_ref[...] = (jnp.dot(h, We2_ref[...], preferred_element_type=jnp.float32)
                   + be2_ref[...])


def _edge_mlp(pre, edge_attr, C, We2, be2_row):
    nblk = E // EBLK
    return pl.pallas_call(
        _edge_mlp_body,
        grid=(nblk,),
        in_specs=[
            pl.BlockSpec((EBLK, H), lambda i: (i, 0)),
            pl.BlockSpec((EBLK, EIN), lambda i: (i, 0)),
            pl.BlockSpec((EIN, H), lambda i: (0, 0)),
            pl.BlockSpec((H, H), lambda i: (0, 0)),
            pl.BlockSpec((1, H), lambda i: (0, 0)),
        ],
        out_specs=pl.BlockSpec((EBLK, H), lambda i: (i, 0)),
        out_shape=jax.ShapeDtypeStruct((E, H), jnp.float32),
    )(pre, edge_attr, C, We2, be2_row)


def _scatter_body(eh_hbm, dst_hbm, out_hbm, didx, rows, shared, sem):
    del sem
    cid = lax.axis_index("c")
    sid = lax.axis_index("s")

    def zero_row(r, c2):
        for k in range(H // 16):
            rows[r, pl.ds(k * 16, 16)] = jnp.zeros((16,), jnp.float32)
        return c2

    lax.fori_loop(0, CH, zero_row, 0)
    for z in range(ZR // CH):
        pltpu.sync_copy(rows, shared.at[pl.ds(sid * ZR + z * CH, CH)])
    pltpu.sync_copy(rows.at[pl.ds(0, ZR % CH)],
                    shared.at[pl.ds(sid * ZR + (ZR // CH) * CH, ZR % CH)])

    @pl.when(sid == 15)
    def _():
        pltpu.sync_copy(rows.at[pl.ds(0, ZTAIL)],
                        shared.at[pl.ds(16 * ZR, ZTAIL)])

    plsc.subcore_barrier()

    base = cid * EPC + sid * EPW

    def chunk(i, carry):
        off = base + i * CH
        pltpu.sync_copy(dst_hbm.at[pl.ds(off, CH)], didx)
        pltpu.sync_copy(eh_hbm.at[pl.ds(off, CH)], rows)
        pltpu.sync_copy(rows, shared.at[didx], add=True)
        return carry

    lax.fori_loop(0, NCH, chunk, 0)
    plsc.subcore_barrier()
    pltpu.sync_copy(shared.at[pl.ds(sid * ZR, ZR)],
                    out_hbm.at[cid, pl.ds(sid * ZR, ZR)])

    @pl.when(sid == 15)
    def _():
        pltpu.sync_copy(shared.at[pl.ds(16 * ZR, ZTAIL)],
                        out_hbm.at[cid, pl.ds(16 * ZR, ZTAIL)])


def _edge_scatter(e_h, dst):
    mesh = plsc.VectorSubcoreMesh(core_axis_name="c", subcore_axis_name="s")
    return pl.kernel(
        _scatter_body,
        out_type=jax.ShapeDtypeStruct((2, N, H), jnp.float32),
        mesh=mesh,
        scratch_types=[
            pltpu.VMEM((CH,), jnp.int32),
            pltpu.VMEM((CH, H), jnp.float32),
            pltpu.VMEM_SHARED((N, H), jnp.float32),
            pltpu.SemaphoreType.DMA,
        ],
    )(e_h, dst)


DROWS = 160
DCOLS = E // DROWS


def _node_global_body(x_ref, u_ref, bcol_ref, brow_ref, aggp_ref, dfl_ref,
                      MkT_ref, Mv_ref, Wn1_ref, bn1_ref, Wn2_ref, bn2_ref,
                      Wg1_ref, bg1_ref, Wg2_ref, bg2_ref,
                      node_out_ref, u_out_ref):
    f32 = jnp.float32
    x = x_ref[...]
    u = u_ref[...]
    agg = aggp_ref[0] + aggp_ref[1]

    gid = lax.broadcasted_iota(jnp.int32, (N, G), 1)
    onehot = (bcol_ref[...] == gid).astype(f32)
    ub = jnp.dot(onehot, u, preferred_element_type=f32)

    logits = (jnp.dot(x, MkT_ref[0:VIN, :], preferred_element_type=f32)
              + jnp.dot(agg, MkT_ref[VIN:2 * VIN, :], preferred_element_type=f32)
              + jnp.dot(ub, MkT_ref[2 * VIN:, :], preferred_element_type=f32))
    m = jnp.max(logits, axis=0, keepdims=True)
    ex = jnp.exp(logits - m)
    attn = ex / jnp.sum(ex, axis=0, keepdims=True)
    attn = attn / jnp.sum(attn, axis=1, keepdims=True)
    attn_out = jnp.dot(attn, Mv_ref[...], preferred_element_type=f32)
    node_out = (jnp.dot(
        jnp.maximum(jnp.dot(attn_out, Wn1_ref[...], preferred_element_type=f32)
                    + bn1_ref[...], 0.0),
        Wn2_ref[...], preferred_element_type=f32) + bn2_ref[...])
    node_out_ref[...] = node_out

    nid = lax.broadcasted_iota(jnp.int32, (G, N), 0)
    onehotT = (nid == brow_ref[...]).astype(f32)
    node_cnt = jnp.sum(onehotT, axis=1, keepdims=True)
    node_agg = (jnp.dot(onehotT, node_out, preferred_element_type=f32)
                / jnp.maximum(node_cnt, 1.0))
    edge_agg_num = jnp.dot(onehotT, agg, preferred_element_type=f32)

    i72 = lax.broadcasted_iota(jnp.int32, (72, G), 0)
    j64 = lax.broadcasted_iota(jnp.int32, (72, G), 1)
    tri = (j64 < i72).astype(f32)
    starts = jnp.dot(tri, node_cnt, preferred_element_type=f32)

    acc = jnp.zeros((72, DCOLS), f32)
    for r in range(DROWS):
        drow = dfl_ref[pl.ds(r, 1), :]
        acc = acc + (drow < starts).astype(f32)
    F = jnp.sum(acc, axis=1, keepdims=True)
    edge_cnt = F[1:G + 1, :] - F[0:G, :]
    edge_agg = edge_agg_num / jnp.maximum(edge_cnt, 1.0)

    gcat = (jnp.dot(u, Wg1_ref[0:UIN, :], preferred_element_type=f32)
            + jnp.dot(node_agg, Wg1_ref[UIN:UIN + H, :], preferred_element_type=f32)
            + jnp.dot(edge_agg, Wg1_ref[UIN + H:, :], preferred_element_type=f32)
            + bg1_ref[...])
    u_out_ref[...] = (jnp.dot(jnp.maximum(gcat, 0.0), Wg2_ref[...],
                              preferred_element_type=f32) + bg2_ref[...])


def _node_global(x, u, batch_col, batch_row, aggp, dfl, MkT, Mv,
                 Wn1, bn1_row, Wn2, bn2_row, Wg1, bg1_row, Wg2, bg2_row):
    return pl.pallas_call(
        _node_global_body,
        out_shape=(jax.ShapeDtypeStruct((N, H), jnp.float32),
                   jax.ShapeDtypeStruct((G, H), jnp.float32)),
    )(x, u, batch_col, batch_row, aggp, dfl, MkT, Mv,
      Wn1, bn1_row, Wn2, bn2_row, Wg1, bg1_row, Wg2, bg2_row)


def kernel(x, edge_index, edge_attr, u, batch, We1, be1, We2, be2, Mk, Mv,
           Wn1, bn1, Wn2, bn2, Wg1, bg1, Wg2, bg2):
    src = edge_index[0]
    dst = edge_index[1]
    batch_col = batch.reshape(N, 1)
    batch_row = batch.reshape(1, N)
    dfl = dst.astype(jnp.float32).reshape(DROWS, DCOLS)

    P2, Q = _node_tables(x, u, batch_col, We1, be1.reshape(1, H))
    pre = _edge_gather(P2, Q, src, dst)
    e_h = _edge_mlp(pre, edge_attr, We1[2 * VIN:2 * VIN + EIN],
                    We2, be2.reshape(1, H))
    aggp = _edge_scatter(e_h, dst)
    node_out, u_out = _node_global(
        x, u, batch_col, batch_row, aggp, dfl, Mk.T, Mv,
        Wn1, bn1.reshape(1, H), Wn2, bn2.reshape(1, H),
        Wg1, bg1.reshape(1, H), Wg2, bg2.reshape(1, H))
    return node_out, e_h, u_out


# speedup vs baseline: 10.6582x; 1.4070x over previous
import functools

import jax
import jax.numpy as jnp
from jax import lax
from jax.experimental import pallas as pl
from jax.experimental.pallas import tpu as pltpu
from jax.experimental.pallas import tpu_sc as plsc

N = 10000
E = 320000
G = 64
VIN = 128
EIN = 16
UIN = 16
H = 128

NW = 32
EPW = E // NW
CH = 80
NCH = EPW // CH
EPC = E // 2
ZR = 624
ZTAIL = N - 16 * ZR


def _tables_body(x_ref, u_ref, batch_ref, We1_ref, be1_ref, P2_ref, Q_ref):
    x = x_ref[...]
    A = We1_ref[0:VIN, :]
    B = We1_ref[VIN:2 * VIN, :]
    D = We1_ref[2 * VIN + EIN:, :]
    uD = jnp.dot(u_ref[...], D, preferred_element_type=jnp.float32)
    gid = lax.broadcasted_iota(jnp.int32, (N, G), 1)
    onehot = (batch_ref[...] == gid).astype(jnp.float32)
    P2_ref[...] = (jnp.dot(x, A, preferred_element_type=jnp.float32)
                   + jnp.dot(onehot, uD, preferred_element_type=jnp.float32)
                   + be1_ref[...])
    Q_ref[...] = jnp.dot(x, B, preferred_element_type=jnp.float32)


def _node_tables(x, u, batch_col, We1, be1_row):
    return pl.pallas_call(
        _tables_body,
        out_shape=(jax.ShapeDtypeStruct((N, H), jnp.float32),
                   jax.ShapeDtypeStruct((N, H), jnp.float32)),
    )(x, u, batch_col, We1, be1_row)


def _gather_body(P2_hbm, Q_hbm, src3_hbm, dst3_hbm, out_hbm,
                 sidx, didx, bufA, bufB, semA, semB, semO):
    wid = lax.axis_index("s") * 2 + lax.axis_index("c")
    base = wid * EPW
    pltpu.sync_copy(src3_hbm.at[wid], sidx)
    pltpu.sync_copy(dst3_hbm.at[wid], didx)

    def start(i, b):
        pltpu.async_copy(P2_hbm.at[sidx.at[i]], bufA[b], semA[b])
        pltpu.async_copy(Q_hbm.at[didx.at[i]], bufB[b], semB[b])

    def finish(i, b):
        pltpu.make_async_copy(P2_hbm.at[sidx.at[i]], bufA[b], semA[b]).wait()
        pltpu.make_async_copy(Q_hbm.at[didx.at[i]], bufB[b], semB[b]).wait()

        def add_row(r, c2):
            for k in range(H // 16):
                v = bufB[b][r, pl.ds(k * 16, 16)]
                plsc.addupdate(bufA[b].at[r, pl.ds(k * 16, 16)], v)
            return c2

        lax.fori_loop(0, CH, add_row, 0)
        pltpu.async_copy(bufA[b], out_hbm.at[pl.ds(base + i * CH, CH)],
                         semO[b])

    def wait_out(i, b):
        pltpu.make_async_copy(
            bufA[b], out_hbm.at[pl.ds(base + i * CH, CH)], semO[b]).wait()

    start(0, 0)
    start(1, 1)

    def pair(jj, carry):
        i = 2 * jj
        finish(i, 0)
        finish(i + 1, 1)

        @pl.when(jj < NCH // 2 - 1)
        def _():
            wait_out(i, 0)
            start(i + 2, 0)
            wait_out(i + 1, 1)
            start(i + 3, 1)

        return carry

    lax.fori_loop(0, NCH // 2, pair, 0)
    wait_out(NCH - 3, 0)
    start(NCH - 1, 0)
    wait_out(NCH - 2, 1)
    finish(NCH - 1, 0)
    wait_out(NCH - 1, 0)


def _edge_gather(P2, Q, src3, dst3):
    mesh = plsc.VectorSubcoreMesh(core_axis_name="c", subcore_axis_name="s")
    return pl.kernel(
        _gather_body,
        out_type=jax.ShapeDtypeStruct((E, H), jnp.float32),
        mesh=mesh,
        scratch_types=[
            pltpu.VMEM((NCH, CH), jnp.int32),
            pltpu.VMEM((NCH, CH), jnp.int32),
            [pltpu.VMEM((CH, H), jnp.float32)] * 2,
            [pltpu.VMEM((CH, H), jnp.float32)] * 2,
            [pltpu.SemaphoreType.DMA] * 2,
            [pltpu.SemaphoreType.DMA] * 2,
            [pltpu.SemaphoreType.DMA] * 2,
        ],
    )(P2, Q, src3, dst3)


EBLK = 2000


def _edge_mlp_body(pre_ref, ea_ref, C_ref, We2_ref, be2_ref, eh_ref):
    h = jnp.maximum(
        pre_ref[...] + jnp.dot(ea_ref[...], C_ref[...],
                               preferred_element_type=jnp.float32), 0.0)
    eh_ref[...] = (jnp.dot(h, We2_ref[...], preferred_element_type=jnp.float32)
                   + be2_ref[...])


def _edge_mlp(pre, edge_attr, C, We2, be2_row):
    nblk = E // EBLK
    return pl.pallas_call(
        _edge_mlp_body,
        grid=(nblk,),
        in_specs=[
            pl.BlockSpec((EBLK, H), lambda i: (i, 0)),
            pl.BlockSpec((EBLK, EIN), lambda i: (i, 0)),
            pl.BlockSpec((EIN, H), lambda i: (0, 0)),
            pl.BlockSpec((H, H), lambda i: (0, 0)),
            pl.BlockSpec((1, H), lambda i: (0, 0)),
        ],
        out_specs=pl.BlockSpec((EBLK, H), lambda i: (i, 0)),
        out_shape=jax.ShapeDtypeStruct((E, H), jnp.float32),
    )(pre, edge_attr, C, We2, be2_row)


def _scatter_body(eh_hbm, dst3_hbm, out_hbm, didx, rows, shared, semR, semS):
    cid = lax.axis_index("c")
    sid = lax.axis_index("s")
    wid = cid * 16 + sid
    base = cid * EPC + sid * EPW
    pltpu.sync_copy(dst3_hbm.at[wid], didx)

    def zero_row(r, c2):
        for k in range(H // 16):
            rows[0][r, pl.ds(k * 16, 16)] = jnp.zeros((16,), jnp.float32)
        return c2

    lax.fori_loop(0, CH, zero_row, 0)
    for z in range(ZR // CH):
        pltpu.sync_copy(rows[0], shared.at[pl.ds(sid * ZR + z * CH, CH)])
    pltpu.sync_copy(rows[0].at[pl.ds(0, ZR % CH)],
                    shared.at[pl.ds(sid * ZR + (ZR // CH) * CH, ZR % CH)])

    @pl.when(sid == 15)
    def _():
        pltpu.sync_copy(rows[0].at[pl.ds(0, ZTAIL)],
                        shared.at[pl.ds(16 * ZR, ZTAIL)])

    plsc.subcore_barrier()

    def start(i, b):
        pltpu.async_copy(eh_hbm.at[pl.ds(base + i * CH, CH)], rows[b],
                         semR[b])

    def finish(i, b):
        pltpu.make_async_copy(eh_hbm.at[pl.ds(base + i * CH, CH)], rows[b],
                              semR[b]).wait()
        pltpu.async_copy(rows[b], shared.at[didx.at[i]], semS[b], add=True)

    def wait_scat(i, b):
        pltpu.make_async_copy(rows[b], shared.at[didx.at[i]],
                              semS[b]).wait()

    start(0, 0)
    start(1, 1)

    def pair(jj, carry):
        i = 2 * jj
        finish(i, 0)
        finish(i + 1, 1)

        @pl.when(jj < NCH // 2 - 1)
        def _():
            wait_scat(i, 0)
            start(i + 2, 0)
            wait_scat(i + 1, 1)
            start(i + 3, 1)

        return carry

    lax.fori_loop(0, NCH // 2, pair, 0)
    wait_scat(NCH - 3, 0)
    start(NCH - 1, 0)
    wait_scat(NCH - 2, 1)
    finish(NCH - 1, 0)
    wait_scat(NCH - 1, 0)

    plsc.subcore_barrier()
    pltpu.sync_copy(shared.at[pl.ds(sid * ZR, ZR)],
                    out_hbm.at[cid, pl.ds(sid * ZR, ZR)])

    @pl.when(sid == 15)
    def _():
        pltpu.sync_copy(shared.at[pl.ds(16 * ZR, ZTAIL)],
                        out_hbm.at[cid, pl.ds(16 * ZR, ZTAIL)])


def _edge_scatter(e_h, dst3):
    mesh = plsc.VectorSubcoreMesh(core_axis_name="c", subcore_axis_name="s")
    return pl.kernel(
        _scatter_body,
        out_type=jax.ShapeDtypeStruct((2, N, H), jnp.float32),
        mesh=mesh,
        scratch_types=[
            pltpu.VMEM((NCH, CH), jnp.int32),
            [pltpu.VMEM((CH, H), jnp.float32)] * 2,
            pltpu.VMEM_SHARED((N, H), jnp.float32),
            [pltpu.SemaphoreType.DMA] * 2,
            [pltpu.SemaphoreType.DMA] * 2,
        ],
    )(e_h, dst3)


DROWS = 160
DCOLS = E // DROWS


def _node_global_body(x_ref, u_ref, bcol_ref, brow_ref, aggp_ref, dfl_ref,
                      MkT_ref, Mv_ref, Wn1_ref, bn1_ref, Wn2_ref, bn2_ref,
                      Wg1_ref, bg1_ref, Wg2_ref, bg2_ref,
                      node_out_ref, u_out_ref):
    f32 = jnp.float32
    x = x_ref[...]
    u = u_ref[...]
    agg = aggp_ref[0] + aggp_ref[1]

    gid = lax.broadcasted_iota(jnp.int32, (N, G), 1)
    onehot = (bcol_ref[...] == gid).astype(f32)
    ub = jnp.dot(onehot, u, preferred_element_type=f32)

    logits = (jnp.dot(x, MkT_ref[0:VIN, :], preferred_element_type=f32)
              + jnp.dot(agg, MkT_ref[VIN:2 * VIN, :], preferred_element_type=f32)
              + jnp.dot(ub, MkT_ref[2 * VIN:, :], preferred_element_type=f32))
    m = jnp.max(logits, axis=0, keepdims=True)
    ex = jnp.exp(logits - m)
    attn = ex / jnp.sum(ex, axis=0, keepdims=True)
    attn = attn / jnp.sum(attn, axis=1, keepdims=True)
    attn_out = jnp.dot(attn, Mv_ref[...], preferred_element_type=f32)
    node_out = (jnp.dot(
        jnp.maximum(jnp.dot(attn_out, Wn1_ref[...], preferred_element_type=f32)
                    + bn1_ref[...], 0.0),
        Wn2_ref[...], preferred_element_type=f32) + bn2_ref[...])
    node_out_ref[...] = node_out

    nid = lax.broadcasted_iota(jnp.int32, (G, N), 0)
    onehotT = (nid == brow_ref[...]).astype(f32)
    node_cnt = jnp.sum(onehotT, axis=1, keepdims=True)
    node_agg = (jnp.dot(onehotT, node_out, preferred_element_type=f32)
                / jnp.maximum(node_cnt, 1.0))
    edge_agg_num = jnp.dot(onehotT, agg, preferred_element_type=f32)

    i72 = lax.broadcasted_iota(jnp.int32, (72, G), 0)
    j64 = lax.broadcasted_iota(jnp.int32, (72, G), 1)
    tri = (j64 < i72).astype(f32)
    starts = jnp.dot(tri, node_cnt, preferred_element_type=f32)

    acc = jnp.zeros((72, DCOLS), f32)
    for r in range(DROWS):
        drow = dfl_ref[pl.ds(r, 1), :]
        acc = acc + (drow < starts).astype(f32)
    F = jnp.sum(acc, axis=1, keepdims=True)
    edge_cnt = F[1:G + 1, :] - F[0:G, :]
    edge_agg = edge_agg_num / jnp.maximum(edge_cnt, 1.0)

    gcat = (jnp.dot(u, Wg1_ref[0:UIN, :], preferred_element_type=f32)
            + jnp.dot(node_agg, Wg1_ref[UIN:UIN + H, :], preferred_element_type=f32)
            + jnp.dot(edge_agg, Wg1_ref[UIN + H:, :], preferred_element_type=f32)
            + bg1_ref[...])
    u_out_ref[...] = (jnp.dot(jnp.maximum(gcat, 0.0), Wg2_ref[...],
                              preferred_element_type=f32) + bg2_ref[...])


def _node_global(x, u, batch_col, batch_row, aggp, dfl, MkT, Mv,
                 Wn1, bn1_row, Wn2, bn2_row, Wg1, bg1_row, Wg2, bg2_row):
    return pl.pallas_call(
        _node_global_body,
        out_shape=(jax.ShapeDtypeStruct((N, H), jnp.float32),
                   jax.ShapeDtypeStruct((G, H), jnp.float32)),
    )(x, u, batch_col, batch_row, aggp, dfl, MkT, Mv,
      Wn1, bn1_row, Wn2, bn2_row, Wg1, bg1_row, Wg2, bg2_row)


def kernel(x, edge_index, edge_attr, u, batch, We1, be1, We2, be2, Mk, Mv,
           Wn1, bn1, Wn2, bn2, Wg1, bg1, Wg2, bg2):
    src = edge_index[0]
    dst = edge_index[1]
    src3 = src.reshape(NW, NCH, CH)
    dst3 = dst.reshape(NW, NCH, CH)
    batch_col = batch.reshape(N, 1)
    batch_row = batch.reshape(1, N)
    dfl = dst.astype(jnp.float32).reshape(DROWS, DCOLS)

    P2, Q = _node_tables(x, u, batch_col, We1, be1.reshape(1, H))
    pre = _edge_gather(P2, Q, src3, dst3)
    e_h = _edge_mlp(pre, edge_attr, We1[2 * VIN:2 * VIN + EIN],
                    We2, be2.reshape(1, H))
    aggp = _edge_scatter(e_h, dst3)
    node_out, u_out = _node_global(
        x, u, batch_col, batch_row, aggp, dfl, Mk.T, Mv,
        Wn1, bn1.reshape(1, H), Wn2, bn2.reshape(1, H),
        Wg1, bg1.reshape(1, H), Wg2, bg2.reshape(1, H))
    return node_out, e_h, u_out


# speedup vs baseline: 10.8160x; 1.0148x over previous
import functools

import jax
import jax.numpy as jnp
from jax import lax
from jax.experimental import pallas as pl
from jax.experimental.pallas import tpu as pltpu
from jax.experimental.pallas import tpu_sc as plsc

N = 10000
E = 320000
G = 64
VIN = 128
EIN = 16
UIN = 16
H = 128

NW = 32
EPW = E // NW
CH = 80
NCH = EPW // CH
EPC = E // 2
ZR = 624
ZTAIL = N - 16 * ZR


def _tables_body(x_ref, u_ref, batch_ref, We1_ref, be1_ref, P2_ref, Q_ref):
    x = x_ref[...]
    A = We1_ref[0:VIN, :]
    B = We1_ref[VIN:2 * VIN, :]
    D = We1_ref[2 * VIN + EIN:, :]
    uD = jnp.dot(u_ref[...], D, preferred_element_type=jnp.float32)
    gid = lax.broadcasted_iota(jnp.int32, (N, G), 1)
    onehot = (batch_ref[...] == gid).astype(jnp.float32)
    P2_ref[...] = (jnp.dot(x, A, preferred_element_type=jnp.float32)
                   + jnp.dot(onehot, uD, preferred_element_type=jnp.float32)
                   + be1_ref[...])
    Q_ref[...] = jnp.dot(x, B, preferred_element_type=jnp.float32)


def _node_tables(x, u, batch_col, We1, be1_row):
    return pl.pallas_call(
        _tables_body,
        out_shape=(jax.ShapeDtypeStruct((N, H), jnp.float32),
                   jax.ShapeDtypeStruct((N, H), jnp.float32)),
    )(x, u, batch_col, We1, be1_row)


def _gather_body(P2_hbm, Q_hbm, src_hbm, dst_hbm, out_hbm,
                 sidx, didx, bufA, bufB, semA, semB, semO):
    wid = lax.axis_index("s") * 2 + lax.axis_index("c")
    base = wid * EPW
    pltpu.sync_copy(src_hbm.at[pl.ds(base, EPW)], sidx)
    pltpu.sync_copy(dst_hbm.at[pl.ds(base, EPW)], didx)

    def start(i, b):
        pltpu.async_copy(P2_hbm.at[sidx.at[pl.ds(i * CH, CH)]], bufA[b],
                         semA[b])
        pltpu.async_copy(Q_hbm.at[didx.at[pl.ds(i * CH, CH)]], bufB[b],
                         semB[b])

    def finish(i, b):
        pltpu.make_async_copy(P2_hbm.at[sidx.at[pl.ds(i * CH, CH)]], bufA[b],
                              semA[b]).wait()
        pltpu.make_async_copy(Q_hbm.at[didx.at[pl.ds(i * CH, CH)]], bufB[b],
                              semB[b]).wait()

        def add_row(r, c2):
            for k in range(H // 16):
                v = bufB[b][r, pl.ds(k * 16, 16)]
                plsc.addupdate(bufA[b].at[r, pl.ds(k * 16, 16)], v)
            return c2

        lax.fori_loop(0, CH, add_row, 0)
        pltpu.async_copy(bufA[b], out_hbm.at[pl.ds(base + i * CH, CH)],
                         semO[b])

    def wait_out(i, b):
        pltpu.make_async_copy(
            bufA[b], out_hbm.at[pl.ds(base + i * CH, CH)], semO[b]).wait()

    start(0, 0)
    start(1, 1)

    def pair(jj, carry):
        i = 2 * jj
        finish(i, 0)
        finish(i + 1, 1)

        @pl.when(jj < NCH // 2 - 1)
        def _():
            wait_out(i, 0)
            start(i + 2, 0)
            wait_out(i + 1, 1)
            start(i + 3, 1)

        return carry

    lax.fori_loop(0, NCH // 2, pair, 0)
    wait_out(NCH - 3, 0)
    start(NCH - 1, 0)
    wait_out(NCH - 2, 1)
    finish(NCH - 1, 0)
    wait_out(NCH - 1, 0)


def _edge_gather(P2, Q, src, dst):
    mesh = plsc.VectorSubcoreMesh(core_axis_name="c", subcore_axis_name="s")
    return pl.kernel(
        _gather_body,
        out_type=jax.ShapeDtypeStruct((E, H), jnp.float32),
        mesh=mesh,
        scratch_types=[
            pltpu.VMEM((EPW,), jnp.int32),
            pltpu.VMEM((EPW,), jnp.int32),
            [pltpu.VMEM((CH, H), jnp.float32)] * 2,
            [pltpu.VMEM((CH, H), jnp.float32)] * 2,
            [pltpu.SemaphoreType.DMA] * 2,
            [pltpu.SemaphoreType.DMA] * 2,
            [pltpu.SemaphoreType.DMA] * 2,
        ],
    )(P2, Q, src, dst)


EBLK = 2000


def _edge_mlp_body(pre_ref, ea_ref, C_ref, We2_ref, be2_ref, eh_ref):
    h = jnp.maximum(
        pre_ref[...] + jnp.dot(ea_ref[...], C_ref[...],
                               preferred_element_type=jnp.float32), 0.0)
    eh_ref[...] = (jnp.dot(h, We2_ref[...], preferred_element_type=jnp.float32)
                   + be2_ref[...])


def _edge_mlp(pre, edge_attr, C, We2, be2_row):
    nblk = E // EBLK
    return pl.pallas_call(
        _edge_mlp_body,
        grid=(nblk,),
        in_specs=[
            pl.BlockSpec((EBLK, H), lambda i: (i, 0)),
            pl.BlockSpec((EBLK, EIN), lambda i: (i, 0)),
            pl.BlockSpec((EIN, H), lambda i: (0, 0)),
            pl.BlockSpec((H, H), lambda i: (0, 0)),
            pl.BlockSpec((1, H), lambda i: (0, 0)),
        ],
        out_specs=pl.BlockSpec((EBLK, H), lambda i: (i, 0)),
        out_shape=jax.ShapeDtypeStruct((E, H), jnp.float32),
    )(pre, edge_attr, C, We2, be2_row)


def _scatter_body(eh_hbm, dst3_hbm, out_hbm, didx, rows, shared, semR, semS):
    cid = lax.axis_index("c")
    sid = lax.axis_index("s")
    wid = cid * 16 + sid
    base = cid * EPC + sid * EPW
    pltpu.sync_copy(dst3_hbm.at[wid], didx)

    def zero_row(r, c2):
        for k in range(H // 16):
            rows[0][r, pl.ds(k * 16, 16)] = jnp.zeros((16,), jnp.float32)
        return c2

    lax.fori_loop(0, CH, zero_row, 0)
    for z in range(ZR // CH):
        pltpu.sync_copy(rows[0], shared.at[pl.ds(sid * ZR + z * CH, CH)])
    pltpu.sync_copy(rows[0].at[pl.ds(0, ZR % CH)],
                    shared.at[pl.ds(sid * ZR + (ZR // CH) * CH, ZR % CH)])

    @pl.when(sid == 15)
    def _():
        pltpu.sync_copy(rows[0].at[pl.ds(0, ZTAIL)],
                        shared.at[pl.ds(16 * ZR, ZTAIL)])

    plsc.subcore_barrier()

    def start(i, b):
        pltpu.async_copy(eh_hbm.at[pl.ds(base + i * CH, CH)], rows[b],
                         semR[b])

    def finish(i, b):
        pltpu.make_async_copy(eh_hbm.at[pl.ds(base + i * CH, CH)], rows[b],
                              semR[b]).wait()
        pltpu.async_copy(rows[b], shared.at[didx.at[i]], semS[b], add=True)

    def wait_scat(i, b):
        pltpu.make_async_copy(rows[b], shared.at[didx.at[i]],
                              semS[b]).wait()

    start(0, 0)
    start(1, 1)

    def pair(jj, carry):
        i = 2 * jj
        finish(i, 0)
        finish(i + 1, 1)

        @pl.when(jj < NCH // 2 - 1)
        def _():
            wait_scat(i, 0)
            start(i + 2, 0)
            wait_scat(i + 1, 1)
            start(i + 3, 1)

        return carry

    lax.fori_loop(0, NCH // 2, pair, 0)
    wait_scat(NCH - 3, 0)
    start(NCH - 1, 0)
    wait_scat(NCH - 2, 1)
    finish(NCH - 1, 0)
    wait_scat(NCH - 1, 0)

    plsc.subcore_barrier()
    pltpu.sync_copy(shared.at[pl.ds(sid * ZR, ZR)],
                    out_hbm.at[cid, pl.ds(sid * ZR, ZR)])

    @pl.when(sid == 15)
    def _():
        pltpu.sync_copy(shared.at[pl.ds(16 * ZR, ZTAIL)],
                        out_hbm.at[cid, pl.ds(16 * ZR, ZTAIL)])


def _edge_scatter(e_h, dst3):
    mesh = plsc.VectorSubcoreMesh(core_axis_name="c", subcore_axis_name="s")
    return pl.kernel(
        _scatter_body,
        out_type=jax.ShapeDtypeStruct((2, N, H), jnp.float32),
        mesh=mesh,
        scratch_types=[
            pltpu.VMEM((NCH, CH), jnp.int32),
            [pltpu.VMEM((CH, H), jnp.float32)] * 2,
            pltpu.VMEM_SHARED((N, H), jnp.float32),
            [pltpu.SemaphoreType.DMA] * 2,
            [pltpu.SemaphoreType.DMA] * 2,
        ],
    )(e_h, dst3)


DROWS = 160
DCOLS = E // DROWS


def _dot_nt(a, b):
    return lax.dot_general(a, b, (((1,), (1,)), ((), ())),
                           preferred_element_type=jnp.float32)


def _node_global_body(x_ref, u_ref, bcol_ref, brow_ref, aggp_ref, dfl_ref,
                      Mk_ref, Mv_ref, Wn1_ref, bn1_ref, Wn2_ref, bn2_ref,
                      Wg1_ref, bg1_ref, Wg2_ref, bg2_ref,
                      node_out_ref, u_out_ref):
    f32 = jnp.float32
    x = x_ref[...]
    u = u_ref[...]
    agg = aggp_ref[0] + aggp_ref[1]

    gid = lax.broadcasted_iota(jnp.int32, (N, G), 1)
    onehot = (bcol_ref[...] == gid).astype(f32)
    ub = jnp.dot(onehot, u, preferred_element_type=f32)

    logits = (_dot_nt(x, Mk_ref[:, 0:VIN])
              + _dot_nt(agg, Mk_ref[:, VIN:2 * VIN])
              + _dot_nt(ub, Mk_ref[:, 2 * VIN:]))
    m = jnp.max(logits, axis=0, keepdims=True)
    ex = jnp.exp(logits - m)
    attn = ex / jnp.sum(ex, axis=0, keepdims=True)
    attn = attn / jnp.sum(attn, axis=1, keepdims=True)
    attn_out = jnp.dot(attn, Mv_ref[...], preferred_element_type=f32)
    node_out = (jnp.dot(
        jnp.maximum(jnp.dot(attn_out, Wn1_ref[...], preferred_element_type=f32)
                    + bn1_ref[...], 0.0),
        Wn2_ref[...], preferred_element_type=f32) + bn2_ref[...])
    node_out_ref[...] = node_out

    nid = lax.broadcasted_iota(jnp.int32, (G, N), 0)
    onehotT = (nid == brow_ref[...]).astype(f32)
    node_cnt = jnp.sum(onehotT, axis=1, keepdims=True)
    node_agg = (jnp.dot(onehotT, node_out, preferred_element_type=f32)
                / jnp.maximum(node_cnt, 1.0))
    edge_agg_num = jnp.dot(onehotT, agg, preferred_element_type=f32)

    i72 = lax.broadcasted_iota(jnp.int32, (72, G), 0)
    j64 = lax.broadcasted_iota(jnp.int32, (72, G), 1)
    tri = (j64 < i72).astype(f32)
    starts = jnp.dot(tri, node_cnt, preferred_element_type=f32)

    acc = jnp.zeros((72, DCOLS), f32)
    for r in range(DROWS):
        drow = dfl_ref[pl.ds(r, 1), :]
        acc = acc + (drow < starts).astype(f32)
    F = jnp.sum(acc, axis=1, keepdims=True)
    edge_cnt = F[1:G + 1, :] - F[0:G, :]
    edge_agg = edge_agg_num / jnp.maximum(edge_cnt, 1.0)

    gcat = (jnp.dot(u, Wg1_ref[0:UIN, :], preferred_element_type=f32)
            + jnp.dot(node_agg, Wg1_ref[UIN:UIN + H, :], preferred_element_type=f32)
            + jnp.dot(edge_agg, Wg1_ref[UIN + H:, :], preferred_element_type=f32)
            + bg1_ref[...])
    u_out_ref[...] = (jnp.dot(jnp.maximum(gcat, 0.0), Wg2_ref[...],
                              preferred_element_type=f32) + bg2_ref[...])


def _node_global(x, u, batch_col, batch_row, aggp, dfl, Mk, Mv,
                 Wn1, bn1_row, Wn2, bn2_row, Wg1, bg1_row, Wg2, bg2_row):
    return pl.pallas_call(
        _node_global_body,
        out_shape=(jax.ShapeDtypeStruct((N, H), jnp.float32),
                   jax.ShapeDtypeStruct((G, H), jnp.float32)),
    )(x, u, batch_col, batch_row, aggp, dfl, Mk, Mv,
      Wn1, bn1_row, Wn2, bn2_row, Wg1, bg1_row, Wg2, bg2_row)


def kernel(x, edge_index, edge_attr, u, batch, We1, be1, We2, be2, Mk, Mv,
           Wn1, bn1, Wn2, bn2, Wg1, bg1, Wg2, bg2):
    src = edge_index[0]
    dst = edge_index[1]
    dst3 = dst.reshape(NW, NCH, CH)
    batch_col = batch.reshape(N, 1)
    batch_row = batch.reshape(1, N)
    dfl = dst.astype(jnp.float32).reshape(DROWS, DCOLS)

    P2, Q = _node_tables(x, u, batch_col, We1, be1.reshape(1, H))
    pre = _edge_gather(P2, Q, src, dst)
    e_h = _edge_mlp(pre, edge_attr, We1[2 * VIN:2 * VIN + EIN],
                    We2, be2.reshape(1, H))
    aggp = _edge_scatter(e_h, dst3)
    node_out, u_out = _node_global(
        x, u, batch_col, batch_row, aggp, dfl, Mk, Mv,
        Wn1, bn1.reshape(1, H), Wn2, bn2.reshape(1, H),
        Wg1, bg1.reshape(1, H), Wg2, bg2.reshape(1, H))
    return node_out, e_h, u_out


# speedup vs baseline: 10.8936x; 1.0072x over previous
import functools

import jax
import jax.numpy as jnp
from jax import lax
from jax.experimental import pallas as pl
from jax.experimental.pallas import tpu as pltpu
from jax.experimental.pallas import tpu_sc as plsc

N = 10000
E = 320000
G = 64
VIN = 128
EIN = 16
UIN = 16
H = 128

NW = 32
EPW = E // NW
CH = 80
NCH = EPW // CH
EPC = E // 2
ZR = 624
ZTAIL = N - 16 * ZR


def _tables_body(x_ref, u_ref, batch_ref, We1_ref, be1_ref, P2_ref, Q_ref):
    x = x_ref[...]
    A = We1_ref[0:VIN, :]
    B = We1_ref[VIN:2 * VIN, :]
    D = We1_ref[2 * VIN + EIN:, :]
    uD = jnp.dot(u_ref[...], D, preferred_element_type=jnp.float32)
    gid = lax.broadcasted_iota(jnp.int32, (N, G), 1)
    onehot = (batch_ref[...] == gid).astype(jnp.float32)
    P2_ref[...] = (jnp.dot(x, A, preferred_element_type=jnp.float32)
                   + jnp.dot(onehot, uD, preferred_element_type=jnp.float32)
                   + be1_ref[...])
    Q_ref[...] = jnp.dot(x, B, preferred_element_type=jnp.float32)


def _node_tables(x, u, batch_col, We1, be1_row):
    return pl.pallas_call(
        _tables_body,
        out_shape=(jax.ShapeDtypeStruct((N, H), jnp.float32),
                   jax.ShapeDtypeStruct((N, H), jnp.float32)),
    )(x, u, batch_col, We1, be1_row)


def _gather_body(P2_hbm, Q_hbm, src_hbm, dst_hbm, out_hbm,
                 sidx, didx, bufA, bufB, semA, semB, semO):
    wid = lax.axis_index("s") * 2 + lax.axis_index("c")
    base = wid * EPW
    pltpu.sync_copy(src_hbm.at[pl.ds(base, EPW)], sidx)
    pltpu.sync_copy(dst_hbm.at[pl.ds(base, EPW)], didx)

    def start(i, b):
        pltpu.async_copy(P2_hbm.at[sidx.at[pl.ds(i * CH, CH)]], bufA[b],
                         semA[b])
        pltpu.async_copy(Q_hbm.at[didx.at[pl.ds(i * CH, CH)]], bufB[b],
                         semB[b])

    def finish(i, b):
        pltpu.make_async_copy(P2_hbm.at[sidx.at[pl.ds(i * CH, CH)]], bufA[b],
                              semA[b]).wait()
        pltpu.make_async_copy(Q_hbm.at[didx.at[pl.ds(i * CH, CH)]], bufB[b],
                              semB[b]).wait()

        def add_row(r2, c2):
            for rr in range(2):
                r = r2 * 2 + rr
                for k in range(H // 16):
                    v = bufB[b][r, pl.ds(k * 16, 16)]
                    plsc.addupdate(bufA[b].at[r, pl.ds(k * 16, 16)], v)
            return c2

        lax.fori_loop(0, CH // 2, add_row, 0)
        pltpu.async_copy(bufA[b], out_hbm.at[pl.ds(base + i * CH, CH)],
                         semO[b])

    def wait_out(i, b):
        pltpu.make_async_copy(
            bufA[b], out_hbm.at[pl.ds(base + i * CH, CH)], semO[b]).wait()

    start(0, 0)
    start(1, 1)
    start(2, 2)

    def triple(jj, carry):
        i = 3 * jj
        finish(i, 0)
        finish(i + 1, 1)
        finish(i + 2, 2)
        for b in range(3):
            @pl.when(i + 3 + b < NCH)
            def _():
                wait_out(i + b, b)
                start(i + 3 + b, b)
        return carry

    lax.fori_loop(0, NCH // 3, triple, 0)
    wait_out(NCH - 3, 2)
    finish(NCH - 2, 0)
    finish(NCH - 1, 1)
    wait_out(NCH - 2, 0)
    wait_out(NCH - 1, 1)


def _edge_gather(P2, Q, src, dst):
    mesh = plsc.VectorSubcoreMesh(core_axis_name="c", subcore_axis_name="s")
    return pl.kernel(
        _gather_body,
        out_type=jax.ShapeDtypeStruct((E, H), jnp.float32),
        mesh=mesh,
        scratch_types=[
            pltpu.VMEM((EPW,), jnp.int32),
            pltpu.VMEM((EPW,), jnp.int32),
            [pltpu.VMEM((CH, H), jnp.float32)] * 3,
            [pltpu.VMEM((CH, H), jnp.float32)] * 3,
            [pltpu.SemaphoreType.DMA] * 3,
            [pltpu.SemaphoreType.DMA] * 3,
            [pltpu.SemaphoreType.DMA] * 3,
        ],
    )(P2, Q, src, dst)


EBLK = 2000


def _edge_mlp_body(pre_ref, ea_ref, C_ref, We2_ref, be2_ref, eh_ref):
    h = jnp.maximum(
        pre_ref[...] + jnp.dot(ea_ref[...], C_ref[...],
                               preferred_element_type=jnp.float32), 0.0)
    eh_ref[...] = (jnp.dot(h, We2_ref[...], preferred_element_type=jnp.float32)
                   + be2_ref[...])


def _edge_mlp(pre, edge_attr, C, We2, be2_row):
    nblk = E // EBLK
    return pl.pallas_call(
        _edge_mlp_body,
        grid=(nblk,),
        in_specs=[
            pl.BlockSpec((EBLK, H), lambda i: (i, 0)),
            pl.BlockSpec((EBLK, EIN), lambda i: (i, 0)),
            pl.BlockSpec((EIN, H), lambda i: (0, 0)),
            pl.BlockSpec((H, H), lambda i: (0, 0)),
            pl.BlockSpec((1, H), lambda i: (0, 0)),
        ],
        out_specs=pl.BlockSpec((EBLK, H), lambda i: (i, 0)),
        out_shape=jax.ShapeDtypeStruct((E, H), jnp.float32),
    )(pre, edge_attr, C, We2, be2_row)


def _scatter_body(eh_hbm, dst3_hbm, out_hbm, didx, rows, shared, semR, semS):
    cid = lax.axis_index("c")
    sid = lax.axis_index("s")
    wid = cid * 16 + sid
    base = cid * EPC + sid * EPW
    pltpu.sync_copy(dst3_hbm.at[wid], didx)

    def zero_row(r, c2):
        for k in range(H // 16):
            rows[0][r, pl.ds(k * 16, 16)] = jnp.zeros((16,), jnp.float32)
        return c2

    lax.fori_loop(0, CH, zero_row, 0)
    for z in range(ZR // CH):
        pltpu.sync_copy(rows[0], shared.at[pl.ds(sid * ZR + z * CH, CH)])
    pltpu.sync_copy(rows[0].at[pl.ds(0, ZR % CH)],
                    shared.at[pl.ds(sid * ZR + (ZR // CH) * CH, ZR % CH)])

    @pl.when(sid == 15)
    def _():
        pltpu.sync_copy(rows[0].at[pl.ds(0, ZTAIL)],
                        shared.at[pl.ds(16 * ZR, ZTAIL)])

    plsc.subcore_barrier()

    def start(i, b):
        pltpu.async_copy(eh_hbm.at[pl.ds(base + i * CH, CH)], rows[b],
                         semR[b])

    def finish(i, b):
        pltpu.make_async_copy(eh_hbm.at[pl.ds(base + i * CH, CH)], rows[b],
                              semR[b]).wait()
        pltpu.async_copy(rows[b], shared.at[didx.at[i]], semS[b], add=True)

    def wait_scat(i, b):
        pltpu.make_async_copy(rows[b], shared.at[didx.at[i]],
                              semS[b]).wait()

    start(0, 0)
    start(1, 1)

    def pair(jj, carry):
        i = 2 * jj
        finish(i, 0)
        finish(i + 1, 1)

        @pl.when(jj < NCH // 2 - 1)
        def _():
            wait_scat(i, 0)
            start(i + 2, 0)
            wait_scat(i + 1, 1)
            start(i + 3, 1)

        return carry

    lax.fori_loop(0, NCH // 2, pair, 0)
    wait_scat(NCH - 3, 0)
    start(NCH - 1, 0)
    wait_scat(NCH - 2, 1)
    finish(NCH - 1, 0)
    wait_scat(NCH - 1, 0)

    plsc.subcore_barrier()
    pltpu.sync_copy(shared.at[pl.ds(sid * ZR, ZR)],
                    out_hbm.at[cid, pl.ds(sid * ZR, ZR)])

    @pl.when(sid == 15)
    def _():
        pltpu.sync_copy(shared.at[pl.ds(16 * ZR, ZTAIL)],
                        out_hbm.at[cid, pl.ds(16 * ZR, ZTAIL)])


def _edge_scatter(e_h, dst3):
    mesh = plsc.VectorSubcoreMesh(core_axis_name="c", subcore_axis_name="s")
    return pl.kernel(
        _scatter_body,
        out_type=jax.ShapeDtypeStruct((2, N, H), jnp.float32),
        mesh=mesh,
        scratch_types=[
            pltpu.VMEM((NCH, CH), jnp.int32),
            [pltpu.VMEM((CH, H), jnp.float32)] * 2,
            pltpu.VMEM_SHARED((N, H), jnp.float32),
            [pltpu.SemaphoreType.DMA] * 2,
            [pltpu.SemaphoreType.DMA] * 2,
        ],
    )(e_h, dst3)


DROWS = 160
DCOLS = E // DROWS


def _dot_nt(a, b):
    return lax.dot_general(a, b, (((1,), (1,)), ((), ())),
                           preferred_element_type=jnp.float32)


def _node_global_body(x_ref, u_ref, bcol_ref, brow_ref, aggp_ref, dfl_ref,
                      Mk_ref, Mv_ref, Wn1_ref, bn1_ref, Wn2_ref, bn2_ref,
                      Wg1_ref, bg1_ref, Wg2_ref, bg2_ref,
                      node_out_ref, u_out_ref):
    f32 = jnp.float32
    x = x_ref[...]
    u = u_ref[...]
    agg = aggp_ref[0] + aggp_ref[1]

    gid = lax.broadcasted_iota(jnp.int32, (N, G), 1)
    onehot = (bcol_ref[...] == gid).astype(f32)
    ub = jnp.dot(onehot, u, preferred_element_type=f32)

    logits = (_dot_nt(x, Mk_ref[:, 0:VIN])
              + _dot_nt(agg, Mk_ref[:, VIN:2 * VIN])
              + _dot_nt(ub, Mk_ref[:, 2 * VIN:]))
    m = jnp.max(logits, axis=0, keepdims=True)
    ex = jnp.exp(logits - m)
    attn = ex / jnp.sum(ex, axis=0, keepdims=True)
    attn = attn / jnp.sum(attn, axis=1, keepdims=True)
    attn_out = jnp.dot(attn, Mv_ref[...], preferred_element_type=f32)
    node_out = (jnp.dot(
        jnp.maximum(jnp.dot(attn_out, Wn1_ref[...], preferred_element_type=f32)
                    + bn1_ref[...], 0.0),
        Wn2_ref[...], preferred_element_type=f32) + bn2_ref[...])
    node_out_ref[...] = node_out

    nid = lax.broadcasted_iota(jnp.int32, (G, N), 0)
    onehotT = (nid == brow_ref[...]).astype(f32)
    node_cnt = jnp.sum(onehotT, axis=1, keepdims=True)
    node_agg = (jnp.dot(onehotT, node_out, preferred_element_type=f32)
                / jnp.maximum(node_cnt, 1.0))
    edge_agg_num = jnp.dot(onehotT, agg, preferred_element_type=f32)

    i72 = lax.broadcasted_iota(jnp.int32, (72, G), 0)
    j64 = lax.broadcasted_iota(jnp.int32, (72, G), 1)
    tri = (j64 < i72).astype(f32)
    starts = jnp.dot(tri, node_cnt, preferred_element_type=f32)

    acc = jnp.zeros((72, DCOLS), f32)
    for r in range(DROWS):
        drow = dfl_ref[pl.ds(r, 1), :]
        acc = acc + (drow < starts).astype(f32)
    F = jnp.sum(acc, axis=1, keepdims=True)
    edge_cnt = F[1:G + 1, :] - F[0:G, :]
    edge_agg = edge_agg_num / jnp.maximum(edge_cnt, 1.0)

    gcat = (jnp.dot(u, Wg1_ref[0:UIN, :], preferred_element_type=f32)
            + jnp.dot(node_agg, Wg1_ref[UIN:UIN + H, :], preferred_element_type=f32)
            + jnp.dot(edge_agg, Wg1_ref[UIN + H:, :], preferred_element_type=f32)
            + bg1_ref[...])
    u_out_ref[...] = (jnp.dot(jnp.maximum(gcat, 0.0), Wg2_ref[...],
                              preferred_element_type=f32) + bg2_ref[...])


def _node_global(x, u, batch_col, batch_row, aggp, dfl, Mk, Mv,
                 Wn1, bn1_row, Wn2, bn2_row, Wg1, bg1_row, Wg2, bg2_row):
    return pl.pallas_call(
        _node_global_body,
        out_shape=(jax.ShapeDtypeStruct((N, H), jnp.float32),
                   jax.ShapeDtypeStruct((G, H), jnp.float32)),
    )(x, u, batch_col, batch_row, aggp, dfl, Mk, Mv,
      Wn1, bn1_row, Wn2, bn2_row, Wg1, bg1_row, Wg2, bg2_row)


def kernel(x, edge_index, edge_attr, u, batch, We1, be1, We2, be2, Mk, Mv,
           Wn1, bn1, Wn2, bn2, Wg1, bg1, Wg2, bg2):
    src = edge_index[0]
    dst = edge_index[1]
    dst3 = dst.reshape(NW, NCH, CH)
    batch_col = batch.reshape(N, 1)
    batch_row = batch.reshape(1, N)
    dfl = dst.astype(jnp.float32).reshape(DROWS, DCOLS)

    P2, Q = _node_tables(x, u, batch_col, We1, be1.reshape(1, H))
    pre = _edge_gather(P2, Q, src, dst)
    e_h = _edge_mlp(pre, edge_attr, We1[2 * VIN:2 * VIN + EIN],
                    We2, be2.reshape(1, H))
    aggp = _edge_scatter(e_h, dst3)
    node_out, u_out = _node_global(
        x, u, batch_col, batch_row, aggp, dfl, Mk, Mv,
        Wn1, bn1.reshape(1, H), Wn2, bn2.reshape(1, H),
        Wg1, bg1.reshape(1, H), Wg2, bg2.reshape(1, H))
    return node_out, e_h, u_out
